# baseline (speedup 1.0000x reference)
import functools

import jax
import jax.numpy as jnp
from jax import lax
from jax.experimental import pallas as pl
from jax.experimental.pallas import tpu as pltpu
from jax.experimental.pallas import tpu_sc as plsc

_N = 10000
_E = 320000
_H = 8
_NCORE = 2
_NSUB = 16
_NW = _NCORE * _NSUB
_EPW = _E // _NW
_K = 200
_BN = 1000
_F32 = jnp.float32

_mesh = plsc.VectorSubcoreMesh(core_axis_name="c", subcore_axis_name="s",
                               num_cores=_NCORE, num_subcores=_NSUB)

_GDN = lax.GatherDimensionNumbers(offset_dims=(), collapsed_slice_dims=(0,),
                                  start_index_map=(0,))


def _vg(vec, idx):
    return lax.gather(vec, idx[:, None], _GDN, (1,),
                      mode=lax.GatherScatterMode.PROMISE_IN_BOUNDS)


def _make_edge_layer(n_et):
    out_type = []
    for _ in range(n_et):
        out_type += [jax.ShapeDtypeStruct((_NCORE, _N, 16), _F32),
                     jax.ShapeDtypeStruct((_NCORE, _N, 64), _F32)]

    @functools.partial(
        pl.kernel,
        out_type=tuple(out_type),
        mesh=_mesh,
        compiler_params=pltpu.CompilerParams(use_tc_tiling_on_sc=False,
                                             needs_layout_passes=False),
        scratch_types=(
            [pltpu.VMEM((_K,), jnp.int32)] * 4
            + [pltpu.VMEM((_K, 16), _F32)] * 6
            + [pltpu.VMEM((_K, 64), _F32)] * 4
            + [pltpu.VMEM_SHARED((_N, 16), _F32),
               pltpu.VMEM_SHARED((_N, 64), _F32)]
            + [pltpu.SemaphoreType.DMA] * 4
        ),
    )
    def edge_layer(*args):
        ins = args[:5 * n_et]
        outs = args[5 * n_et:7 * n_et]
        (si0, si1, di0, di1, as0, as1, ad0, ad1, ex0, ex1,
         hb0, hb1, mb0, mb1, den_sh, agg_sh, sg0, sg1, ss0, ss1) = args[7 * n_et:]
        cid = lax.axis_index("c")
        sid = lax.axis_index("s")
        wid = sid * _NCORE + cid
        ii = lax.iota(jnp.int32, 16)
        perms = [jnp.where(ii < 8, 2 * g, 2 * g + 1) for g in range(4)]
        si_v, di_v = (si0, si1), (di0, di1)
        asb, adb, exb = (as0, as1), (ad0, ad1), (ex0, ex1)
        hb, mb = (hb0, hb1), (mb0, mb1)
        sg, ss = (sg0, sg1), (ss0, ss1)
        nc = _EPW // _K

        for e in range(n_et):
            tsrc, tdst, hsrc, si, di = ins[5 * e:5 * e + 5]
            den_o, agg_o = outs[2 * e:2 * e + 2]

            @pl.when(sid == 0)
            def _zero():
                def zrow(i, _):
                    ex0[i, :] = jnp.zeros((16,), _F32)
                    for g in range(4):
                        mb0[i, pl.ds(16 * g, 16)] = jnp.zeros((16,), _F32)
                    return 0
                lax.fori_loop(0, _K, zrow, 0)

                def zchunk(c, _):
                    pltpu.sync_copy(ex0, den_sh.at[pl.ds(c * _K, _K), :])
                    pltpu.sync_copy(mb0, agg_sh.at[pl.ds(c * _K, _K), :])
                    return 0
                lax.fori_loop(0, _N // _K, zchunk, 0)

            plsc.subcore_barrier()

            def g_issue(c, b, tsrc=tsrc, tdst=tdst, hsrc=hsrc, si=si, di=di):
                base = pl.multiple_of(wid * _EPW + c * _K, 8)
                pltpu.sync_copy(si.at[pl.ds(base, _K)], si_v[b])
                pltpu.sync_copy(di.at[pl.ds(base, _K)], di_v[b])
                pltpu.async_copy(tsrc.at[si_v[b]], asb[b], sg[b])
                pltpu.async_copy(tdst.at[di_v[b]], adb[b], sg[b])
                pltpu.async_copy(hsrc.at[si_v[b]], hb[b], sg[b])

            def g_wait(b, tsrc=tsrc, tdst=tdst, hsrc=hsrc):
                pltpu.make_async_copy(tsrc.at[si_v[b]], asb[b], sg[b]).wait()
                pltpu.make_async_copy(tdst.at[di_v[b]], adb[b], sg[b]).wait()
                pltpu.make_async_copy(hsrc.at[si_v[b]], hb[b], sg[b]).wait()

            def s_issue(b):
                pltpu.async_copy(exb[b], den_sh.at[di_v[b]], ss[b], add=True)
                pltpu.async_copy(mb[b], agg_sh.at[di_v[b]], ss[b], add=True)

            def s_wait(b):
                pltpu.make_async_copy(exb[b], den_sh.at[di_v[b]], ss[b]).wait()
                pltpu.make_async_copy(mb[b], agg_sh.at[di_v[b]], ss[b]).wait()

            g_issue(0, 0)

            def outer(j, _):
                for b in range(2):
                    c = j * 2 + b
                    g_wait(b)

                    @pl.when(c >= 1)
                    def _drain():
                        s_wait(1 - b)

                    @pl.when(c + 1 < nc)
                    def _nxt():
                        g_issue(c + 1, 1 - b)

                    @plsc.parallel_loop(0, _K, unroll=8)
                    def row(i):
                        x = asb[b][i, :] + adb[b][i, :]
                        ex = jnp.exp(jnp.maximum(x, 0.2 * x))
                        exb[b][i, :] = ex
                        for g in range(4):
                            mb[b][i, pl.ds(16 * g, 16)] = (
                                hb[b][i, pl.ds(16 * g, 16)] * _vg(ex, perms[g]))
                    s_issue(b)
                return 0
            lax.fori_loop(0, nc // 2, outer, 0)
            s_wait(1)

            plsc.subcore_barrier()

            @pl.when(sid == 0)
            def _out():
                pltpu.sync_copy(den_sh, den_o.at[cid])
                pltpu.sync_copy(agg_sh, agg_o.at[cid])

    return edge_layer


_edge_layer3 = _make_edge_layer(3)
_edge_layer2 = _make_edge_layer(2)


def _relu(x):
    return jnp.maximum(x, 0.0)


def _ln(x, g, b):
    mu = jnp.mean(x, axis=-1, keepdims=True)
    var = jnp.var(x, axis=-1, keepdims=True)
    return (x - mu) / jnp.sqrt(var + 1e-5) * g + b


def _stage1(xa, xt, wa, ba, wt, bt, att_stack, sel):
    fin = xa.shape[1]
    t = len(sel)

    def body(xa_r, xt_r, wa_r, ba_r, wt_r, bt_r, att_r, ha_o, ht_o, *tab_o):
        ha = jnp.dot(xa_r[...], wa_r[...], preferred_element_type=_F32) + ba_r[...]
        ht = jnp.dot(xt_r[...], wt_r[...], preferred_element_type=_F32) + bt_r[...]
        ha_o[...] = ha
        ht_o[...] = ht
        for j, s in enumerate(sel):
            h = ht if s else ha
            tab_o[j][...] = jnp.dot(h, att_r[j], preferred_element_type=_F32)

    full = lambda shape: pl.BlockSpec(shape, lambda i: (0,) * len(shape))
    return pl.pallas_call(
        body,
        grid=(_N // _BN,),
        in_specs=[
            pl.BlockSpec((_BN, fin), lambda i: (i, 0)),
            pl.BlockSpec((_BN, fin), lambda i: (i, 0)),
            full((fin, 64)), full((1, 64)), full((fin, 64)), full((1, 64)),
            full((t, 64, 16)),
        ],
        out_specs=[pl.BlockSpec((_BN, 64), lambda i: (i, 0))] * 2
        + [pl.BlockSpec((_BN, 16), lambda i: (i, 0))] * t,
        out_shape=[jax.ShapeDtypeStruct((_N, 64), _F32)] * 2
        + [jax.ShapeDtypeStruct((_N, 16), _F32)] * t,
    )(xa, xt, wa, ba, wt, bt, att_stack)


def _stage2a(rels, rexp, klw, klb):
    k = len(rels)

    def body(*refs):
        dens = refs[0:2 * k:2]
        aggs = refs[1:2 * k:2]
        rexp_r, klw_r, klb_r = refs[2 * k:2 * k + 3]
        xs_o = refs[2 * k + 3:3 * k + 3]
        cs_o = refs[3 * k + 3]
        i = pl.program_id(0)
        cols = []
        for j in range(k):
            dinv = 1.0 / (dens[j][0] + dens[j][1] + 1e-16)
            scale = jnp.dot(dinv, rexp_r[...], preferred_element_type=_F32)
            xs = _relu(aggs[j][0] + aggs[j][1]) * scale
            xs_o[j][...] = xs
            if j < 2:
                kx = jnp.tanh(jnp.dot(xs, klw_r[...],
                                      preferred_element_type=_F32) + klb_r[...])
                cols.append(jnp.sum(kx, axis=0, keepdims=True))
        s = jnp.concatenate(cols, axis=0)

        @pl.when(i == 0)
        def _init():
            cs_o[...] = s

        @pl.when(i > 0)
        def _acc():
            cs_o[...] += s

    dpart = pl.BlockSpec((2, _BN, 16), lambda i: (0, i, 0))
    apart = pl.BlockSpec((2, _BN, 64), lambda i: (0, i, 0))
    full = lambda shape: pl.BlockSpec(shape, lambda i: (0,) * len(shape))
    nb = pl.BlockSpec((_BN, 64), lambda i: (i, 0))
    ins, in_specs = [], []
    for den, agg in rels:
        ins += [den, agg]
        in_specs += [dpart, apart]
    ins += [rexp, klw, klb]
    in_specs += [full((16, 64)), full((64, 64)), full((1, 64))]
    out_specs = [nb] * k + [pl.BlockSpec((2, 64), lambda i: (0, 0))]
    out_shape = ([jax.ShapeDtypeStruct((_N, 64), _F32)] * k
                 + [jax.ShapeDtypeStruct((2, 64), _F32)])
    return pl.pallas_call(body, grid=(_N // _BN,), in_specs=in_specs,
                          out_specs=out_specs, out_shape=out_shape)(*ins)


def _beta(cs_ref, q_ref):
    s0 = jnp.sum(cs_ref[0:1, :] * q_ref[...]) / _N
    s1 = jnp.sum(cs_ref[1:2, :] * q_ref[...]) / _N
    m = jnp.maximum(s0, s1)
    e0 = jnp.exp(s0 - m)
    e1 = jnp.exp(s1 - m)
    return e0 / (e0 + e1), e1 / (e0 + e1)


def _stage2b_l1(xs_ta, xs_aa, xs_at, cs, q, g, b):
    def body(xta, xaa, xat, cs_r, q_r, g_r, b_r, ha_o, ht_o):
        b0, b1 = _beta(cs_r, q_r)
        out_a = b0 * xta[...] + b1 * xaa[...]
        ha_o[...] = _relu(_ln(out_a, g_r[...], b_r[...]))
        ht_o[...] = _relu(_ln(xat[...], g_r[...], b_r[...]))

    nb = pl.BlockSpec((_BN, 64), lambda i: (i, 0))
    full = lambda shape: pl.BlockSpec(shape, lambda i: (0,) * len(shape))
    return pl.pallas_call(
        body,
        grid=(_N // _BN,),
        in_specs=[nb, nb, nb, full((2, 64)), full((1, 64)), full((1, 64)),
                  full((1, 64))],
        out_specs=[nb, nb],
        out_shape=[jax.ShapeDtypeStruct((_N, 64), _F32)] * 2,
    )(xs_ta, xs_aa, xs_at, cs, q, g, b)


def _stage2b_l2(xs_ta, xs_aa, cs, q, g, b, lw, lb):
    def body(xta, xaa, cs_r, q_r, g_r, b_r, lw_r, lb_r, o):
        b0, b1 = _beta(cs_r, q_r)
        h = _relu(_ln(b0 * xta[...] + b1 * xaa[...], g_r[...], b_r[...]))
        o[...] = jnp.dot(h, lw_r[...], preferred_element_type=_F32) + lb_r[...]

    nb = pl.BlockSpec((_BN, 64), lambda i: (i, 0))
    full = lambda shape: pl.BlockSpec(shape, lambda i: (0,) * len(shape))
    return pl.pallas_call(
        body,
        grid=(_N // _BN,),
        in_specs=[nb, nb, full((2, 64)), full((1, 64)), full((1, 64)),
                  full((1, 64)), full((64, 2)), full((1, 2))],
        out_specs=pl.BlockSpec((_BN, 2), lambda i: (i, 0)),
        out_shape=jax.ShapeDtypeStruct((_N, 2), _F32),
    )(xs_ta, xs_aa, cs, q, g, b, lw, lb)


def _a16(att):
    flat = att.reshape(64)
    return jnp.zeros((64, 16), _F32).at[jnp.arange(64), jnp.arange(64) // _H].set(flat)


def kernel(x_address, x_transaction, params, edge_index_at, edge_index_ta,
           edge_index_aa):
    p = params
    edges = {'at': edge_index_at, 'ta': edge_index_ta, 'aa': edge_index_aa}
    rel = {'at': (0, 1), 'ta': (1, 0), 'aa': (0, 0)}
    rexp = jnp.zeros((16, 64), _F32).at[jnp.arange(64) // _H, jnp.arange(64)].set(1.0)

    def han_layer(prefix, xa, xt, ens):
        sel, atts = [], []
        for en in ens:
            s, d = rel[en]
            sel += [s, d]
            atts += [_a16(p[prefix + '_att_src_' + en]),
                     _a16(p[prefix + '_att_dst_' + en])]
        outs = _stage1(xa, xt,
                       p[prefix + '_W_address'], p[prefix + '_b_address'].reshape(1, 64),
                       p[prefix + '_W_transaction'], p[prefix + '_b_transaction'].reshape(1, 64),
                       jnp.stack(atts), sel)
        h = {0: outs[0], 1: outs[1]}
        tabs = {en: (outs[2 + 2 * j], outs[3 + 2 * j]) for j, en in enumerate(ens)}

        ins = []
        for en in ens:
            s, _d = rel[en]
            ins += [tabs[en][0], tabs[en][1], h[s], edges[en][0], edges[en][1]]
        fn = _edge_layer3 if len(ens) == 3 else _edge_layer2
        outs2 = fn(*ins)
        return [(outs2[2 * j], outs2[2 * j + 1]) for j in range(len(ens))]

    rels1 = han_layer('c1', x_address, x_transaction, ['ta', 'aa', 'at'])
    xs_ta, xs_aa, xs_at, cs1 = _stage2a(rels1, rexp, p['c1_klin_W'],
                                        p['c1_klin_b'].reshape(1, 64))
    h1a, h1t = _stage2b_l1(xs_ta, xs_aa, xs_at, cs1, p['c1_q'].reshape(1, 64),
                           p['ln1_g'].reshape(1, 64), p['ln1_b'].reshape(1, 64))

    rels2 = han_layer('c2', h1a, h1t, ['ta', 'aa'])
    xs2_ta, xs2_aa, cs2 = _stage2a(rels2, rexp, p['c2_klin_W'],
                                   p['c2_klin_b'].reshape(1, 64))
    return _stage2b_l2(xs2_ta, xs2_aa, cs2, p['c2_q'].reshape(1, 64),
                       p['ln2_g'].reshape(1, 64), p['ln2_b'].reshape(1, 64),
                       p['lin_W'], p['lin_b'].reshape(1, 2))


# speedup vs baseline: 1.0754x; 1.0754x over previous
import functools

import jax
import jax.numpy as jnp
from jax import lax
from jax.experimental import pallas as pl
from jax.experimental.pallas import tpu as pltpu
from jax.experimental.pallas import tpu_sc as plsc

_N = 10000
_E = 320000
_H = 8
_NCORE = 2
_NSUB = 16
_NW = _NCORE * _NSUB
_EPW = _E // _NW
_K = 200
_BN = 1000
_F32 = jnp.float32

_mesh = plsc.VectorSubcoreMesh(core_axis_name="c", subcore_axis_name="s",
                               num_cores=_NCORE, num_subcores=_NSUB)

_GDN = lax.GatherDimensionNumbers(offset_dims=(), collapsed_slice_dims=(0,),
                                  start_index_map=(0,))


def _vg(vec, idx):
    return lax.gather(vec, idx[:, None], _GDN, (1,),
                      mode=lax.GatherScatterMode.PROMISE_IN_BOUNDS)


def _make_edge_layer(n_et):
    out_type = []
    for _ in range(n_et):
        out_type += [jax.ShapeDtypeStruct((_NCORE, _N, 16), _F32),
                     jax.ShapeDtypeStruct((_NCORE, _N, 64), _F32)]

    @functools.partial(
        pl.kernel,
        out_type=tuple(out_type),
        mesh=_mesh,
        compiler_params=pltpu.CompilerParams(use_tc_tiling_on_sc=False,
                                             needs_layout_passes=False),
        scratch_types=(
            [pltpu.VMEM((_K,), jnp.int32)] * 4
            + [pltpu.VMEM((_K, 16), _F32)] * 6
            + [pltpu.VMEM((_K, 64), _F32)] * 4
            + [pltpu.VMEM_SHARED((_N, 16), _F32),
               pltpu.VMEM_SHARED((_N, 64), _F32)]
            + [pltpu.SemaphoreType.DMA] * 4
        ),
    )
    def edge_layer(*args):
        ins = args[:5 * n_et]
        outs = args[5 * n_et:7 * n_et]
        (si0, si1, di0, di1, as0, as1, ad0, ad1, ex0, ex1,
         hb0, hb1, mb0, mb1, den_sh, agg_sh, sg0, sg1, ss0, ss1) = args[7 * n_et:]
        cid = lax.axis_index("c")
        sid = lax.axis_index("s")
        wid = sid * _NCORE + cid
        ii = lax.iota(jnp.int32, 16)
        perms = [jnp.where(ii < 8, 2 * g, 2 * g + 1) for g in range(4)]
        si_v, di_v = (si0, si1), (di0, di1)
        asb, adb, exb = (as0, as1), (ad0, ad1), (ex0, ex1)
        hb, mb = (hb0, hb1), (mb0, mb1)
        sg, ss = (sg0, sg1), (ss0, ss1)
        nc = _EPW // _K

        for e in range(n_et):
            tsrc, tdst, hsrc, si, di = ins[5 * e:5 * e + 5]
            den_o, agg_o = outs[2 * e:2 * e + 2]

            @pl.when(sid == 0)
            def _zero():
                def zrow(i, _):
                    ex0[i, :] = jnp.zeros((16,), _F32)
                    for g in range(4):
                        mb0[i, pl.ds(16 * g, 16)] = jnp.zeros((16,), _F32)
                    return 0
                lax.fori_loop(0, _K, zrow, 0)

                def zchunk(c, _):
                    pltpu.sync_copy(ex0, den_sh.at[pl.ds(c * _K, _K), :])
                    pltpu.sync_copy(mb0, agg_sh.at[pl.ds(c * _K, _K), :])
                    return 0
                lax.fori_loop(0, _N // _K, zchunk, 0)

            plsc.subcore_barrier()

            def g_issue(c, b, tsrc=tsrc, tdst=tdst, hsrc=hsrc, si=si, di=di):
                base = pl.multiple_of(wid * _EPW + c * _K, 8)
                pltpu.sync_copy(si.at[pl.ds(base, _K)], si_v[b])
                pltpu.sync_copy(di.at[pl.ds(base, _K)], di_v[b])
                pltpu.async_copy(tsrc.at[si_v[b]], asb[b], sg[b])
                pltpu.async_copy(tdst.at[di_v[b]], adb[b], sg[b])
                pltpu.async_copy(hsrc.at[si_v[b]], hb[b], sg[b])

            def g_wait(b, tsrc=tsrc, tdst=tdst, hsrc=hsrc):
                pltpu.make_async_copy(tsrc.at[si_v[b]], asb[b], sg[b]).wait()
                pltpu.make_async_copy(tdst.at[di_v[b]], adb[b], sg[b]).wait()
                pltpu.make_async_copy(hsrc.at[si_v[b]], hb[b], sg[b]).wait()

            def s_issue(b):
                pltpu.async_copy(exb[b], den_sh.at[di_v[b]], ss[b], add=True)
                pltpu.async_copy(mb[b], agg_sh.at[di_v[b]], ss[b], add=True)

            def s_wait(b):
                pltpu.make_async_copy(exb[b], den_sh.at[di_v[b]], ss[b]).wait()
                pltpu.make_async_copy(mb[b], agg_sh.at[di_v[b]], ss[b]).wait()

            g_issue(0, 0)

            def outer(j, _):
                for b in range(2):
                    c = j * 2 + b
                    g_wait(b)

                    @pl.when(c >= 1)
                    def _drain():
                        s_wait(1 - b)

                    @pl.when(c + 1 < nc)
                    def _nxt():
                        g_issue(c + 1, 1 - b)

                    @plsc.parallel_loop(0, _K, unroll=8)
                    def row(i):
                        x = asb[b][i, :] + adb[b][i, :]
                        ex = jnp.exp(jnp.maximum(x, 0.2 * x))
                        exb[b][i, :] = ex
                        for g in range(4):
                            mb[b][i, pl.ds(16 * g, 16)] = (
                                hb[b][i, pl.ds(16 * g, 16)] * _vg(ex, perms[g]))
                    s_issue(b)
                return 0
            lax.fori_loop(0, nc // 2, outer, 0)
            s_wait(1)

            plsc.subcore_barrier()

            @pl.when(sid == 0)
            def _out():
                pltpu.sync_copy(den_sh, den_o.at[cid])
                pltpu.sync_copy(agg_sh, agg_o.at[cid])

    return edge_layer


_edge_layer1 = _make_edge_layer(1)


def _relu(x):
    return jnp.maximum(x, 0.0)


def _ln(x, g, b):
    mu = jnp.mean(x, axis=-1, keepdims=True)
    var = jnp.var(x, axis=-1, keepdims=True)
    return (x - mu) / jnp.sqrt(var + 1e-5) * g + b


def _stage1(xa, xt, wa, ba, wt, bt, att_stack, sel):
    fin = xa.shape[1]
    t = len(sel)

    def body(xa_r, xt_r, wa_r, ba_r, wt_r, bt_r, att_r, ha_o, ht_o, *tab_o):
        ha = jnp.dot(xa_r[...], wa_r[...], preferred_element_type=_F32) + ba_r[...]
        ht = jnp.dot(xt_r[...], wt_r[...], preferred_element_type=_F32) + bt_r[...]
        ha_o[...] = ha
        ht_o[...] = ht
        for j, s in enumerate(sel):
            h = ht if s else ha
            tab_o[j][...] = jnp.dot(h, att_r[j], preferred_element_type=_F32)

    full = lambda shape: pl.BlockSpec(shape, lambda i: (0,) * len(shape))
    return pl.pallas_call(
        body,
        grid=(_N // _BN,),
        in_specs=[
            pl.BlockSpec((_BN, fin), lambda i: (i, 0)),
            pl.BlockSpec((_BN, fin), lambda i: (i, 0)),
            full((fin, 64)), full((1, 64)), full((fin, 64)), full((1, 64)),
            full((t, 64, 16)),
        ],
        out_specs=[pl.BlockSpec((_BN, 64), lambda i: (i, 0))] * 2
        + [pl.BlockSpec((_BN, 16), lambda i: (i, 0))] * t,
        out_shape=[jax.ShapeDtypeStruct((_N, 64), _F32)] * 2
        + [jax.ShapeDtypeStruct((_N, 16), _F32)] * t,
    )(xa, xt, wa, ba, wt, bt, att_stack)


def _stage2a(rels, rexp, klw, klb):
    k = len(rels)

    def body(*refs):
        dens = refs[0:2 * k:2]
        aggs = refs[1:2 * k:2]
        rexp_r, klw_r, klb_r = refs[2 * k:2 * k + 3]
        xs_o = refs[2 * k + 3:3 * k + 3]
        cs_o = refs[3 * k + 3]
        i = pl.program_id(0)
        cols = []
        for j in range(k):
            dinv = 1.0 / (dens[j][0] + dens[j][1] + 1e-16)
            scale = jnp.dot(dinv, rexp_r[...], preferred_element_type=_F32)
            xs = _relu(aggs[j][0] + aggs[j][1]) * scale
            xs_o[j][...] = xs
            if j < 2:
                kx = jnp.tanh(jnp.dot(xs, klw_r[...],
                                      preferred_element_type=_F32) + klb_r[...])
                cols.append(jnp.sum(kx, axis=0, keepdims=True))
        s = jnp.concatenate(cols, axis=0)

        @pl.when(i == 0)
        def _init():
            cs_o[...] = s

        @pl.when(i > 0)
        def _acc():
            cs_o[...] += s

    dpart = pl.BlockSpec((2, _BN, 16), lambda i: (0, i, 0))
    apart = pl.BlockSpec((2, _BN, 64), lambda i: (0, i, 0))
    full = lambda shape: pl.BlockSpec(shape, lambda i: (0,) * len(shape))
    nb = pl.BlockSpec((_BN, 64), lambda i: (i, 0))
    ins, in_specs = [], []
    for den, agg in rels:
        ins += [den, agg]
        in_specs += [dpart, apart]
    ins += [rexp, klw, klb]
    in_specs += [full((16, 64)), full((64, 64)), full((1, 64))]
    out_specs = [nb] * k + [pl.BlockSpec((2, 64), lambda i: (0, 0))]
    out_shape = ([jax.ShapeDtypeStruct((_N, 64), _F32)] * k
                 + [jax.ShapeDtypeStruct((2, 64), _F32)])
    return pl.pallas_call(body, grid=(_N // _BN,), in_specs=in_specs,
                          out_specs=out_specs, out_shape=out_shape)(*ins)


def _beta(cs_ref, q_ref):
    s0 = jnp.sum(cs_ref[0:1, :] * q_ref[...]) / _N
    s1 = jnp.sum(cs_ref[1:2, :] * q_ref[...]) / _N
    m = jnp.maximum(s0, s1)
    e0 = jnp.exp(s0 - m)
    e1 = jnp.exp(s1 - m)
    return e0 / (e0 + e1), e1 / (e0 + e1)


def _stage2b_l1(xs_ta, xs_aa, xs_at, cs, q, g, b):
    def body(xta, xaa, xat, cs_r, q_r, g_r, b_r, ha_o, ht_o):
        b0, b1 = _beta(cs_r, q_r)
        out_a = b0 * xta[...] + b1 * xaa[...]
        ha_o[...] = _relu(_ln(out_a, g_r[...], b_r[...]))
        ht_o[...] = _relu(_ln(xat[...], g_r[...], b_r[...]))

    nb = pl.BlockSpec((_BN, 64), lambda i: (i, 0))
    full = lambda shape: pl.BlockSpec(shape, lambda i: (0,) * len(shape))
    return pl.pallas_call(
        body,
        grid=(_N // _BN,),
        in_specs=[nb, nb, nb, full((2, 64)), full((1, 64)), full((1, 64)),
                  full((1, 64))],
        out_specs=[nb, nb],
        out_shape=[jax.ShapeDtypeStruct((_N, 64), _F32)] * 2,
    )(xs_ta, xs_aa, xs_at, cs, q, g, b)


def _stage2b_l2(xs_ta, xs_aa, cs, q, g, b, lw, lb):
    def body(xta, xaa, cs_r, q_r, g_r, b_r, lw_r, lb_r, o):
        b0, b1 = _beta(cs_r, q_r)
        h = _relu(_ln(b0 * xta[...] + b1 * xaa[...], g_r[...], b_r[...]))
        o[...] = jnp.dot(h, lw_r[...], preferred_element_type=_F32) + lb_r[...]

    nb = pl.BlockSpec((_BN, 64), lambda i: (i, 0))
    full = lambda shape: pl.BlockSpec(shape, lambda i: (0,) * len(shape))
    return pl.pallas_call(
        body,
        grid=(_N // _BN,),
        in_specs=[nb, nb, full((2, 64)), full((1, 64)), full((1, 64)),
                  full((1, 64)), full((64, 2)), full((1, 2))],
        out_specs=pl.BlockSpec((_BN, 2), lambda i: (i, 0)),
        out_shape=jax.ShapeDtypeStruct((_N, 2), _F32),
    )(xs_ta, xs_aa, cs, q, g, b, lw, lb)


def _a16(att):
    flat = att.reshape(64)
    return jnp.zeros((64, 16), _F32).at[jnp.arange(64), jnp.arange(64) // _H].set(flat)


def kernel(x_address, x_transaction, params, edge_index_at, edge_index_ta,
           edge_index_aa):
    p = params
    edges = {'at': edge_index_at, 'ta': edge_index_ta, 'aa': edge_index_aa}
    rel = {'at': (0, 1), 'ta': (1, 0), 'aa': (0, 0)}
    rexp = jnp.zeros((16, 64), _F32).at[jnp.arange(64) // _H, jnp.arange(64)].set(1.0)

    def han_layer(prefix, xa, xt, ens):
        sel, atts = [], []
        for en in ens:
            s, d = rel[en]
            sel += [s, d]
            atts += [_a16(p[prefix + '_att_src_' + en]),
                     _a16(p[prefix + '_att_dst_' + en])]
        outs = _stage1(xa, xt,
                       p[prefix + '_W_address'], p[prefix + '_b_address'].reshape(1, 64),
                       p[prefix + '_W_transaction'], p[prefix + '_b_transaction'].reshape(1, 64),
                       jnp.stack(atts), sel)
        h = {0: outs[0], 1: outs[1]}
        tabs = {en: (outs[2 + 2 * j], outs[3 + 2 * j]) for j, en in enumerate(ens)}

        rels = []
        for en in ens:
            s, _d = rel[en]
            den, agg = _edge_layer1(tabs[en][0], tabs[en][1], h[s],
                                    edges[en][0], edges[en][1])
            rels.append((den, agg))
        return rels

    rels1 = han_layer('c1', x_address, x_transaction, ['ta', 'aa', 'at'])
    xs_ta, xs_aa, xs_at, cs1 = _stage2a(rels1, rexp, p['c1_klin_W'],
                                        p['c1_klin_b'].reshape(1, 64))
    h1a, h1t = _stage2b_l1(xs_ta, xs_aa, xs_at, cs1, p['c1_q'].reshape(1, 64),
                           p['ln1_g'].reshape(1, 64), p['ln1_b'].reshape(1, 64))

    rels2 = han_layer('c2', h1a, h1t, ['ta', 'aa'])
    xs2_ta, xs2_aa, cs2 = _stage2a(rels2, rexp, p['c2_klin_W'],
                                   p['c2_klin_b'].reshape(1, 64))
    return _stage2b_l2(xs2_ta, xs2_aa, cs2, p['c2_q'].reshape(1, 64),
                       p['ln2_g'].reshape(1, 64), p['ln2_b'].reshape(1, 64),
                       p['lin_W'], p['lin_b'].reshape(1, 2))
